# baseline (speedup 1.0000x reference)
import jax
import jax.numpy as jnp
from jax.experimental import pallas as pl
from jax.experimental.pallas import tpu as pltpu

_N = 10000
_CHUNK = 2000
_NC = _N // _CHUNK
_SPLIT = 2
_SUB = _CHUNK // _SPLIT


def _mlp_kernel(z_hbm, w1_ref, b1_ref, w2_ref, b2_ref, out_hbm,
                z_vmem, o_vmem, load_sem, store_sem):
    def load(i):
        rows = pl.ds(i * _CHUNK, _CHUNK)
        return pltpu.make_async_copy(z_hbm.at[rows, :], z_vmem.at[rows, :],
                                     load_sem.at[i])

    def store(i):
        rows = pl.ds(i * _SUB, _SUB)
        return pltpu.make_async_copy(o_vmem.at[rows, :], out_hbm.at[rows, :],
                                     store_sem.at[i])

    for i in range(_NC):
        load(i).start()
    for i in range(_NC):
        load(i).wait()
        for j in range(_SPLIT):
            rows = pl.ds(i * _CHUNK + j * _SUB, _SUB)
            h = jnp.dot(z_vmem[rows, :], w1_ref[...],
                        preferred_element_type=jnp.float32)
            h = jnp.maximum(h + b1_ref[...], 0.0)
            o = jnp.dot(h, w2_ref[...], preferred_element_type=jnp.float32)
            o_vmem[rows, :] = o + b2_ref[...]
            store(i * _SPLIT + j).start()
    for i in range(_NC * _SPLIT):
        store(i).wait()


def kernel(z, edge_index, W1, b1, W2, b2):
    n, k = z.shape
    hdim = W1.shape[1]
    odim = W2.shape[1]
    out = pl.pallas_call(
        _mlp_kernel,
        in_specs=[
            pl.BlockSpec(memory_space=pltpu.MemorySpace.HBM),
            pl.BlockSpec(memory_space=pltpu.VMEM),
            pl.BlockSpec(memory_space=pltpu.VMEM),
            pl.BlockSpec(memory_space=pltpu.VMEM),
            pl.BlockSpec(memory_space=pltpu.VMEM),
        ],
        out_specs=pl.BlockSpec(memory_space=pltpu.MemorySpace.HBM),
        out_shape=jax.ShapeDtypeStruct((n, odim), jnp.float32),
        scratch_shapes=[
            pltpu.VMEM((n, k), jnp.float32),
            pltpu.VMEM((n, odim), jnp.float32),
            pltpu.SemaphoreType.DMA((_NC,)),
            pltpu.SemaphoreType.DMA((_NC * _SPLIT,)),
        ],
    )(z, W1, b1.reshape(1, hdim), W2, b2.reshape(1, odim))
    return out.reshape(-1)


# speedup vs baseline: 1.0194x; 1.0194x over previous
import jax
import jax.numpy as jnp
from jax.experimental import pallas as pl
from jax.experimental.pallas import tpu as pltpu

_N = 10000
_CHUNK = 2000
_NC = _N // _CHUNK
_GROUP = 2
_NS = _NC // _GROUP
_SUB = _CHUNK * _GROUP


def _mlp_kernel(z_hbm, w1_ref, b1_ref, w2_ref, b2_ref, out_hbm,
                z_vmem, o_vmem, load_sem, store_sem):
    def load(i):
        rows = pl.ds(i * _CHUNK, _CHUNK)
        return pltpu.make_async_copy(z_hbm.at[rows, :], z_vmem.at[rows, :],
                                     load_sem.at[i])

    def store(i):
        rows = pl.ds(i * _SUB, _SUB)
        return pltpu.make_async_copy(o_vmem.at[rows, :], out_hbm.at[rows, :],
                                     store_sem.at[i])

    for i in range(_NC):
        load(i).start()
    for i in range(_NC):
        load(i).wait()
        rows = pl.ds(i * _CHUNK, _CHUNK)
        h = jnp.dot(z_vmem[rows, :], w1_ref[...],
                    preferred_element_type=jnp.float32)
        h = jnp.maximum(h + b1_ref[...], 0.0)
        o = jnp.dot(h, w2_ref[...], preferred_element_type=jnp.float32)
        o_vmem[rows, :] = o + b2_ref[...]
        if (i + 1) % _GROUP == 0:
            store(i // _GROUP).start()
    for i in range(_NS):
        store(i).wait()


def kernel(z, edge_index, W1, b1, W2, b2):
    n, k = z.shape
    hdim = W1.shape[1]
    odim = W2.shape[1]
    out = pl.pallas_call(
        _mlp_kernel,
        in_specs=[
            pl.BlockSpec(memory_space=pltpu.MemorySpace.HBM),
            pl.BlockSpec(memory_space=pltpu.VMEM),
            pl.BlockSpec(memory_space=pltpu.VMEM),
            pl.BlockSpec(memory_space=pltpu.VMEM),
            pl.BlockSpec(memory_space=pltpu.VMEM),
        ],
        out_specs=pl.BlockSpec(memory_space=pltpu.MemorySpace.HBM),
        out_shape=jax.ShapeDtypeStruct((n, odim), jnp.float32),
        scratch_shapes=[
            pltpu.VMEM((n, k), jnp.float32),
            pltpu.VMEM((n, odim), jnp.float32),
            pltpu.SemaphoreType.DMA((_NC,)),
            pltpu.SemaphoreType.DMA((_NS,)),
        ],
    )(z, W1, b1.reshape(1, hdim), W2, b2.reshape(1, odim))
    return out.reshape(-1)
